# baseline (speedup 1.0000x reference)
import functools

import jax
import jax.numpy as jnp
from jax import lax
from jax.experimental import pallas as pl
from jax.experimental.pallas import tpu as pltpu
from jax.experimental.pallas import tpu_sc as plsc

ROWS = 8192
DIM = 1024
NUM_CORES = 2
NUM_SUBCORES = 16
NUM_WORKERS = NUM_CORES * NUM_SUBCORES
ROWS_PER_WORKER = ROWS // NUM_WORKERS
NCHUNK = 4
CHUNK = ROWS_PER_WORKER // NCHUNK

_mesh = plsc.VectorSubcoreMesh(core_axis_name="c", subcore_axis_name="s")


@functools.partial(
    pl.kernel,
    mesh=_mesh,
    out_type=jax.ShapeDtypeStruct((ROWS, DIM), jnp.float32),
    scratch_types=[pltpu.SemaphoreType.DMA] * NCHUNK,
)
def _sc_copy(src_hbm, out_hbm, *sems):
    wid = lax.axis_index("s") * NUM_CORES + lax.axis_index("c")
    base = wid * ROWS_PER_WORKER

    def cp(g):
        sl_ = pl.ds(base + g * CHUNK, CHUNK)
        return pltpu.make_async_copy(src_hbm.at[sl_], out_hbm.at[sl_], sems[g])

    for g in range(NCHUNK):
        cp(g).start()
    for g in range(NCHUNK):
        cp(g).wait()


def kernel(x, emb_weight):
    sl = x.shape[1]
    out = _sc_copy(emb_weight)
    return out[:sl]


# speedup vs baseline: 21.6682x; 21.6682x over previous
import functools

import jax
import jax.numpy as jnp
from jax import lax
from jax.experimental import pallas as pl
from jax.experimental.pallas import tpu as pltpu
from jax.experimental.pallas import tpu_sc as plsc

ROWS = 8192
DIM = 1024
NUM_CORES = 2
NUM_SUBCORES = 16
NUM_WORKERS = NUM_CORES * NUM_SUBCORES
ACTIVE_WORKERS = 16
ROWS_PER_WORKER = ROWS // ACTIVE_WORKERS
NBUF = 3
CHUNK = 32
NCHUNK = ROWS_PER_WORKER // CHUNK
SLACK = 1

_mesh = plsc.VectorSubcoreMesh(core_axis_name="c", subcore_axis_name="s")


@functools.partial(
    pl.kernel,
    mesh=_mesh,
    out_type=jax.ShapeDtypeStruct((ROWS, DIM), jnp.float32),
    scratch_types=(
        [pltpu.VMEM((CHUNK, DIM), jnp.float32)] * NBUF
        + [pltpu.SemaphoreType.DMA] * (2 * NBUF)
    ),
)
def _sc_copy(src_hbm, out_hbm, *scratch):
    bufs = scratch[:NBUF]
    in_sems = scratch[NBUF:2 * NBUF]
    out_sems = scratch[2 * NBUF:]

    wid = lax.axis_index("s") * NUM_CORES + lax.axis_index("c")
    base = wid * ROWS_PER_WORKER

    def cp_in(g, b):
        return pltpu.make_async_copy(
            src_hbm.at[pl.ds(base + g * CHUNK, CHUNK)], bufs[b], in_sems[b])

    def cp_out(g, b):
        return pltpu.make_async_copy(
            bufs[b], out_hbm.at[pl.ds(base + g * CHUNK, CHUNK)], out_sems[b])

    @pl.when(wid < ACTIVE_WORKERS)
    def _():
        for b in range(min(NBUF, NCHUNK)):
            cp_in(b, b).start()
        waited_out = set()
        for g in range(NCHUNK):
            nxt = g + NBUF - SLACK
            if NBUF <= nxt < NCHUNK:
                prev = nxt - NBUF
                cp_out(prev, prev % NBUF).wait()
                waited_out.add(prev)
                cp_in(nxt, nxt % NBUF).start()
            cp_in(g, g % NBUF).wait()
            cp_out(g, g % NBUF).start()
        for g in range(NCHUNK):
            if g not in waited_out:
                cp_out(g, g % NBUF).wait()


def kernel(x, emb_weight):
    sl = x.shape[1]
    out = _sc_copy(emb_weight)
    return out[:sl]


# speedup vs baseline: 24.8997x; 1.1491x over previous
import functools

import jax
import jax.numpy as jnp
from jax import lax
from jax.experimental import pallas as pl
from jax.experimental.pallas import tpu as pltpu
from jax.experimental.pallas import tpu_sc as plsc

ROWS = 8192
DIM = 1024
NUM_CORES = 2
NUM_SUBCORES = 16
NUM_WORKERS = NUM_CORES * NUM_SUBCORES
ROWS_PER_WORKER = ROWS // NUM_WORKERS
NBUF = 3
CHUNK = 32
NCHUNK = ROWS_PER_WORKER // CHUNK
SLACK = 1

_mesh = plsc.VectorSubcoreMesh(core_axis_name="c", subcore_axis_name="s")


@functools.partial(
    pl.kernel,
    mesh=_mesh,
    out_type=jax.ShapeDtypeStruct((ROWS, DIM), jnp.float32),
    scratch_types=(
        [pltpu.VMEM((CHUNK, DIM), jnp.float32)] * NBUF
        + [pltpu.SemaphoreType.DMA] * (2 * NBUF)
    ),
)
def _sc_copy(src_hbm, out_hbm, *scratch):
    bufs = scratch[:NBUF]
    in_sems = scratch[NBUF:2 * NBUF]
    out_sems = scratch[2 * NBUF:]

    wid = lax.axis_index("s") * NUM_CORES + lax.axis_index("c")
    base = wid * ROWS_PER_WORKER

    def cp_in(g, b):
        return pltpu.make_async_copy(
            src_hbm.at[pl.ds(base + g * CHUNK, CHUNK)], bufs[b], in_sems[b])

    def cp_out(g, b):
        return pltpu.make_async_copy(
            bufs[b], out_hbm.at[pl.ds(base + g * CHUNK, CHUNK)], out_sems[b])

    for b in range(min(NBUF, NCHUNK)):
        cp_in(b, b).start()
    waited_out = set()
    for g in range(NCHUNK):
        nxt = g + NBUF - SLACK
        if NBUF <= nxt < NCHUNK:
            prev = nxt - NBUF
            cp_out(prev, prev % NBUF).wait()
            waited_out.add(prev)
            cp_in(nxt, nxt % NBUF).start()
        cp_in(g, g % NBUF).wait()
        cp_out(g, g % NBUF).start()
    for g in range(NCHUNK):
        if g not in waited_out:
            cp_out(g, g % NBUF).wait()


def kernel(x, emb_weight):
    sl = x.shape[1]
    out = _sc_copy(emb_weight)
    return out[:sl]


# speedup vs baseline: 24.9704x; 1.0028x over previous
import functools

import jax
import jax.numpy as jnp
from jax import lax
from jax.experimental import pallas as pl
from jax.experimental.pallas import tpu as pltpu
from jax.experimental.pallas import tpu_sc as plsc

ROWS = 8192
DIM = 1024
NUM_CORES = 2
NUM_SUBCORES = 16
NUM_WORKERS = NUM_CORES * NUM_SUBCORES
ROWS_PER_WORKER = ROWS // NUM_WORKERS
NBUF = 3
CHUNK = 32
NCHUNK = ROWS_PER_WORKER // CHUNK
SLACK = 1

_mesh = plsc.VectorSubcoreMesh(core_axis_name="c", subcore_axis_name="s")


@functools.partial(
    pl.kernel,
    mesh=_mesh,
    out_type=jax.ShapeDtypeStruct((ROWS, DIM), jnp.float32),
    scratch_types=(
        [pltpu.VMEM((CHUNK, DIM), jnp.float32)] * NBUF
        + [pltpu.SemaphoreType.DMA] * (2 * NBUF)
    ),
)
def _sc_copy(src_hbm, out_hbm, *scratch):
    bufs = scratch[:NBUF]
    in_sems = scratch[NBUF:2 * NBUF]
    out_sems = scratch[2 * NBUF:]

    wid = lax.axis_index("c") * NUM_SUBCORES + lax.axis_index("s")
    base = wid * ROWS_PER_WORKER

    def cp_in(g, b):
        return pltpu.make_async_copy(
            src_hbm.at[pl.ds(base + g * CHUNK, CHUNK)], bufs[b], in_sems[b])

    def cp_out(g, b):
        return pltpu.make_async_copy(
            bufs[b], out_hbm.at[pl.ds(base + g * CHUNK, CHUNK)], out_sems[b])

    for b in range(min(NBUF, NCHUNK)):
        cp_in(b, b).start()
    waited_out = set()
    for g in range(NCHUNK):
        nxt = g + NBUF - SLACK
        if NBUF <= nxt < NCHUNK:
            prev = nxt - NBUF
            cp_out(prev, prev % NBUF).wait()
            waited_out.add(prev)
            cp_in(nxt, nxt % NBUF).start()
        cp_in(g, g % NBUF).wait()
        cp_out(g, g % NBUF).start()
    for g in range(NCHUNK):
        if g not in waited_out:
            cp_out(g, g % NBUF).wait()


def kernel(x, emb_weight):
    sl = x.shape[1]
    out = _sc_copy(emb_weight)
    return out[:sl]
